# baseline (speedup 1.0000x reference)
import jax
import jax.numpy as jnp
from jax import lax
from jax.experimental import pallas as pl
from jax.experimental.pallas import tpu as pltpu
from jax.experimental.pallas import tpu_sc as plsc

NUM_USERS = 1000000
NUM_FEATURE = 26
EMBED_K = 16
BATCH = 16384

NC = 2
NS = 16
L = 16
NW = NC * NS
BPW = BATCH // NW
NGRP = BPW // L


def _sc_body(u_hbm, v_hbm, ft_hbm, wt_hbm, ht_hbm, p_hbm, out1_hbm, out2_hbm,
             uidx, vidx, fbuf, pbuf, ublk, vblk,
             o1buf, o2buf, sem_f, sem_u, sem_v):
    wid = lax.axis_index("s") * NC + lax.axis_index("c")
    base = wid * BPW

    f_cps = [
        pltpu.async_copy(ft_hbm.at[k].at[pl.ds(base, BPW)],
                         fbuf.at[pl.ds(k * BPW, BPW)], sem_f)
        for k in range(NUM_FEATURE)
    ]
    pltpu.sync_copy(u_hbm.at[pl.ds(base, BPW)], uidx)
    pltpu.sync_copy(v_hbm.at[pl.ds(base, BPW)], vidx)
    pltpu.sync_copy(p_hbm, pbuf)
    for cp in f_cps:
        cp.wait()

    lanes = lax.iota(jnp.int32, L)
    dwa = plsc.load_gather(pbuf, [lanes])
    dwb = plsc.load_gather(pbuf, [lanes + L])
    dbv = plsc.load_gather(pbuf, [lanes + 2 * L])
    mask = jnp.full((L,), 127, jnp.int32)
    nmask = jnp.int32(~127)
    lo8 = jnp.bitwise_and(lanes, 7)
    lomask = lanes < 8
    HC = L // 2

    def fire(h):
        s0 = jnp.bitwise_and(h, 1) * HC
        jvec8 = h * HC + lo8
        uvec = plsc.load_gather(uidx, [jvec8])
        vvec = plsc.load_gather(vidx, [jvec8])
        for j in range(HC):
            offu = pl.multiple_of(jnp.bitwise_and(uvec[j], nmask), 128)
            offv = pl.multiple_of(jnp.bitwise_and(vvec[j], nmask), 128)
            pltpu.async_copy(wt_hbm.at[:, :, pl.ds(offu, 128)],
                             ublk.at[s0 + j], sem_u)
            pltpu.async_copy(ht_hbm.at[:, :, pl.ds(offv, 128)],
                             vblk.at[s0 + j], sem_v)

    def drain():
        for j in range(HC):
            pltpu.make_async_copy(wt_hbm.at[:, :, pl.ds(0, 128)],
                                  ublk.at[j], sem_u).wait()
            pltpu.make_async_copy(ht_hbm.at[:, :, pl.ds(0, 128)],
                                  vblk.at[j], sem_v).wait()

    def compute(h):
        s0 = jnp.bitwise_and(h, 1) * HC
        jvec8 = h * HC + lo8
        uvec = plsc.load_gather(uidx, [jvec8])
        vvec = plsc.load_gather(vidx, [jvec8])
        cu = jnp.bitwise_and(uvec, mask)
        cv = jnp.bitwise_and(vvec, mask)
        slot = s0 + lo8
        acc1 = jnp.zeros((L,), jnp.float32)
        tr2 = lanes // 8
        for k in range(8):
            rs = jnp.full((L,), k, jnp.int32)
            gu = plsc.load_gather(ublk, [slot, tr2, rs, cu])
            gv = plsc.load_gather(vblk, [slot, tr2, rs, cv])
            acc1 = acc1 + gu * gv
        acc2 = dbv
        for k in range(NUM_FEATURE):
            gf = plsc.load_gather(fbuf, [jvec8 + k * BPW])
            wk = dwa[k] if k < L else dwb[k - L]
            acc2 = acc2 + gf * wk
        o2 = jnp.minimum(jnp.maximum(jnp.exp(acc2), 1e-05), 3.0)
        plsc.store_scatter(o1buf, [jvec8], acc1, mask=lomask)
        plsc.addupdate_scatter(o1buf, [jvec8], acc1,
                               mask=jnp.logical_not(lomask))
        plsc.store_scatter(o2buf, [jvec8], o2, mask=lomask)

    NH = 2 * NGRP

    def step(h, _):
        drain()
        fire(h)
        compute(h - 1)
        return _

    fire(jnp.int32(0))
    lax.fori_loop(1, NH, step, None)
    drain()
    compute(jnp.int32(NH - 1))

    pltpu.sync_copy(o1buf, out1_hbm.at[pl.ds(base, BPW)])
    pltpu.sync_copy(o2buf, out2_hbm.at[pl.ds(base, BPW)])


@jax.jit
def kernel(x, feature, W, H, Dw, Db):
    u_idx = x[:, 0]
    v_idx = x[:, 1]
    ft = feature.T
    wt = W.T.reshape(2, 8, NUM_USERS)
    ht = H.T.reshape(2, 8, NUM_USERS)
    params = jnp.concatenate([
        Dw[:, 0],
        jnp.zeros((2 * L - NUM_FEATURE,), jnp.float32),
        jnp.broadcast_to(Db, (L,)),
    ])
    mesh = plsc.VectorSubcoreMesh(core_axis_name="c", subcore_axis_name="s",
                                  num_cores=NC, num_subcores=NS)
    out1, out2 = pl.kernel(
        _sc_body,
        out_type=[
            jax.ShapeDtypeStruct((BATCH,), jnp.float32),
            jax.ShapeDtypeStruct((BATCH,), jnp.float32),
        ],
        mesh=mesh,
        compiler_params=pltpu.CompilerParams(needs_layout_passes=False,
                                             use_tc_tiling_on_sc=True),
        scratch_types=[
            pltpu.VMEM((BPW,), jnp.int32),
            pltpu.VMEM((BPW,), jnp.int32),
            pltpu.VMEM((NUM_FEATURE * BPW,), jnp.float32),
            pltpu.VMEM((3 * L,), jnp.float32),
            pltpu.VMEM((L, 2, 8, 128), jnp.float32),
            pltpu.VMEM((L, 2, 8, 128), jnp.float32),
            pltpu.VMEM((BPW,), jnp.float32),
            pltpu.VMEM((BPW,), jnp.float32),
            pltpu.SemaphoreType.DMA,
            pltpu.SemaphoreType.DMA,
            pltpu.SemaphoreType.DMA,
        ],
    )(u_idx, v_idx, ft, wt, ht, params)
    return out1, out2


# speedup vs baseline: 1.0709x; 1.0709x over previous
import jax
import jax.numpy as jnp
from jax import lax
from jax.experimental import pallas as pl
from jax.experimental.pallas import tpu as pltpu
from jax.experimental.pallas import tpu_sc as plsc

NUM_USERS = 1000000
NUM_FEATURE = 26
EMBED_K = 16
BATCH = 16384

NC = 2
NS = 16
L = 16
NW = NC * NS
BPW = BATCH // NW
NGRP = BPW // L


def _sc_body(u_hbm, v_hbm, ft_hbm, wt_hbm, ht_hbm, p_hbm, out1_hbm, out2_hbm,
             uidx, vidx, fbuf, pbuf, ublk, vblk,
             o1buf, o2buf, sem_f, sem_u, sem_v):
    wid = lax.axis_index("s") * NC + lax.axis_index("c")
    base = wid * BPW

    f_cps = [
        pltpu.async_copy(ft_hbm.at[k].at[pl.ds(base, BPW)],
                         fbuf.at[pl.ds(k * BPW, BPW)], sem_f)
        for k in range(NUM_FEATURE)
    ]
    pltpu.sync_copy(u_hbm.at[pl.ds(base, BPW)], uidx)
    pltpu.sync_copy(v_hbm.at[pl.ds(base, BPW)], vidx)
    pltpu.sync_copy(p_hbm, pbuf)
    for cp in f_cps:
        cp.wait()

    lanes = lax.iota(jnp.int32, L)
    dwa = plsc.load_gather(pbuf, [lanes])
    dwb = plsc.load_gather(pbuf, [lanes + L])
    dbv = plsc.load_gather(pbuf, [lanes + 2 * L])
    mask = jnp.full((L,), 127, jnp.int32)
    nmask = jnp.int32(~127)

    def chunk(g, _):
        jvec = g * L + lanes
        uvec = plsc.load_gather(uidx, [jvec])
        vvec = plsc.load_gather(vidx, [jvec])
        cu = jnp.bitwise_and(uvec, mask)
        cv = jnp.bitwise_and(vvec, mask)
        u_cps = []
        v_cps = []
        for j in range(L):
            offu = pl.multiple_of(jnp.bitwise_and(uvec[j], nmask), 128)
            offv = pl.multiple_of(jnp.bitwise_and(vvec[j], nmask), 128)
            u_cps.append(pltpu.async_copy(
                wt_hbm.at[:, :, pl.ds(offu, 128)], ublk.at[j], sem_u))
            v_cps.append(pltpu.async_copy(
                ht_hbm.at[:, :, pl.ds(offv, 128)], vblk.at[j], sem_v))
        for cp in u_cps:
            cp.wait()
        for cp in v_cps:
            cp.wait()

        acc1 = jnp.zeros((L,), jnp.float32)
        for k in range(EMBED_K):
            trs = jnp.full((L,), k // 8, jnp.int32)
            rs = jnp.full((L,), k % 8, jnp.int32)
            gu = plsc.load_gather(ublk, [lanes, trs, rs, cu])
            gv = plsc.load_gather(vblk, [lanes, trs, rs, cv])
            acc1 = acc1 + gu * gv
        acc2 = dbv
        for k in range(NUM_FEATURE):
            gf = plsc.load_gather(fbuf, [jvec + k * BPW])
            wk = dwa[k] if k < L else dwb[k - L]
            acc2 = acc2 + gf * wk
        o2 = jnp.minimum(jnp.maximum(jnp.exp(acc2), 1e-05), 3.0)
        plsc.store_scatter(o1buf, [jvec], acc1)
        plsc.store_scatter(o2buf, [jvec], o2)
        return _

    lax.fori_loop(0, NGRP, chunk, None)

    pltpu.sync_copy(o1buf, out1_hbm.at[pl.ds(base, BPW)])
    pltpu.sync_copy(o2buf, out2_hbm.at[pl.ds(base, BPW)])


@jax.jit
def kernel(x, feature, W, H, Dw, Db):
    u_idx = x[:, 0]
    v_idx = x[:, 1]
    ft = feature.T
    wt = W.T.reshape(2, 8, NUM_USERS)
    ht = H.T.reshape(2, 8, NUM_USERS)
    params = jnp.concatenate([
        Dw[:, 0],
        jnp.zeros((2 * L - NUM_FEATURE,), jnp.float32),
        jnp.broadcast_to(Db, (L,)),
    ])
    mesh = plsc.VectorSubcoreMesh(core_axis_name="c", subcore_axis_name="s",
                                  num_cores=NC, num_subcores=NS)
    out1, out2 = pl.kernel(
        _sc_body,
        out_type=[
            jax.ShapeDtypeStruct((BATCH,), jnp.float32),
            jax.ShapeDtypeStruct((BATCH,), jnp.float32),
        ],
        mesh=mesh,
        compiler_params=pltpu.CompilerParams(needs_layout_passes=False,
                                             use_tc_tiling_on_sc=True),
        scratch_types=[
            pltpu.VMEM((BPW,), jnp.int32),
            pltpu.VMEM((BPW,), jnp.int32),
            pltpu.VMEM((NUM_FEATURE * BPW,), jnp.float32),
            pltpu.VMEM((3 * L,), jnp.float32),
            pltpu.VMEM((L, 2, 8, 128), jnp.float32),
            pltpu.VMEM((L, 2, 8, 128), jnp.float32),
            pltpu.VMEM((BPW,), jnp.float32),
            pltpu.VMEM((BPW,), jnp.float32),
            pltpu.SemaphoreType.DMA,
            pltpu.SemaphoreType.DMA,
            pltpu.SemaphoreType.DMA,
        ],
    )(u_idx, v_idx, ft, wt, ht, params)
    return out1, out2
